# baseline (speedup 1.0000x reference)
import functools

import jax
import jax.numpy as jnp
from jax import lax
from jax.experimental import pallas as pl
from jax.experimental.pallas import tpu as pltpu
from jax.experimental.pallas import tpu_sc as plsc

B, C, D = 16384, 100, 64
NC, NS, L = 2, 16, 16
NW = NC * NS
BPW = B // NW
CHUNK = 4
NBUF = 4
NCHUNK = BPW // CHUNK
NGRP = NCHUNK // NBUF
DK = D // L

_mesh = plsc.VectorSubcoreMesh(core_axis_name="c", subcore_axis_name="s")


@functools.partial(
    pl.kernel,
    mesh=_mesh,
    out_type=jax.ShapeDtypeStruct((B, C, D), jnp.float32),
    scratch_types=[
        pltpu.VMEM((C, D), jnp.float32),
        pltpu.VMEM((NBUF, CHUNK, C, D), jnp.float32),
        pltpu.SemaphoreType.DMA((NBUF,)),
        pltpu.SemaphoreType.DMA((NBUF,)),
    ],
    compiler_params=pltpu.CompilerParams(use_tc_tiling_on_sc=False),
)
def _col_add(x_hbm, t_hbm, o_hbm, tbuf, bufs, sin, sout):
    wid = lax.axis_index("s") * NC + lax.axis_index("c")
    base = wid * BPW
    pltpu.sync_copy(t_hbm, tbuf)

    def start_in(chunk_idx, b):
        pltpu.async_copy(
            x_hbm.at[pl.ds(base + chunk_idx * CHUNK, CHUNK)],
            bufs.at[b], sin.at[b])

    def wait_in(b):
        pltpu.make_async_copy(
            x_hbm.at[pl.ds(base, CHUNK)], bufs.at[b], sin.at[b]).wait()

    def start_out(chunk_idx, b):
        pltpu.async_copy(
            bufs.at[b],
            o_hbm.at[pl.ds(base + chunk_idx * CHUNK, CHUNK)], sout.at[b])

    def wait_out(b):
        pltpu.make_async_copy(
            bufs.at[b], o_hbm.at[pl.ds(base, CHUNK)], sout.at[b]).wait()

    def compute(b):
        @plsc.parallel_loop(0, C, unroll=2)
        def _(c):
            for k in range(DK):
                sl = pl.ds(k * L, L)
                t = tbuf[c, sl]
                for r in range(CHUNK):
                    bufs[b, r, c, sl] += t

    start_in(0, 0)
    start_in(1, 1)

    def group(g, carry):
        for b in range(NBUF):
            i = g * NBUF + b
            bp = (b + 2) % NBUF

            @pl.when(i + 2 < NCHUNK)
            def _():
                @pl.when(i >= 2)
                def _():
                    wait_out(bp)
                start_in(i + 2, bp)

            wait_in(b)
            compute(b)
            start_out(i, b)
        return carry

    lax.fori_loop(0, NGRP, group, 0)
    for b in range(NBUF):
        wait_out(b)


def kernel(inputs, table):
    return _col_add(inputs, table)


# speedup vs baseline: 7.1764x; 7.1764x over previous
import functools

import jax
import jax.numpy as jnp
from jax import lax
from jax.experimental import pallas as pl
from jax.experimental.pallas import tpu as pltpu
from jax.experimental.pallas import tpu_sc as plsc

B, C, D = 16384, 100, 64
NC, NS, L = 2, 16, 16
NW = NC * NS
BPW = B // NW
HD = D // 2
NCHUNK = C * 2
NBUF = 4
NGRP = NCHUNK // NBUF
KG = BPW // L

_mesh = plsc.VectorSubcoreMesh(core_axis_name="c", subcore_axis_name="s")


@functools.partial(
    pl.kernel,
    mesh=_mesh,
    out_type=jax.ShapeDtypeStruct((C, D, B), jnp.float32),
    scratch_types=[
        pltpu.VMEM((C, D), jnp.float32),
        pltpu.VMEM((HD, L), jnp.float32),
        pltpu.VMEM((NBUF, HD, BPW), jnp.float32),
        pltpu.SemaphoreType.DMA((NBUF,)),
        pltpu.SemaphoreType.DMA((NBUF,)),
    ],
    compiler_params=pltpu.CompilerParams(use_tc_tiling_on_sc=True),
)
def _col_add(x_hbm, t_hbm, o_hbm, tbuf, texp, bufs, sin, sout):
    wid = lax.axis_index("s") * NC + lax.axis_index("c")
    b0 = wid * BPW
    pltpu.sync_copy(t_hbm, tbuf)

    def start_in(i, b):
        c, h = i // 2, (i % 2) * HD
        pltpu.async_copy(
            x_hbm.at[c, pl.ds(h, HD), pl.ds(b0, BPW)], bufs.at[b], sin.at[b])

    def wait_in(b):
        pltpu.make_async_copy(
            x_hbm.at[0, pl.ds(0, HD), pl.ds(b0, BPW)], bufs.at[b],
            sin.at[b]).wait()

    def start_out(i, b):
        c, h = i // 2, (i % 2) * HD
        pltpu.async_copy(
            bufs.at[b], o_hbm.at[c, pl.ds(h, HD), pl.ds(b0, BPW)], sout.at[b])

    def wait_out(b):
        pltpu.make_async_copy(
            bufs.at[b], o_hbm.at[0, pl.ds(0, HD), pl.ds(b0, BPW)],
            sout.at[b]).wait()

    def compute(i, b):
        c, h = i // 2, (i % 2) * HD
        for g in range(HD // L):
            tv = tbuf[c, pl.ds(h + g * L, L)]
            for j in range(L):
                texp[g * L + j, :] = jnp.broadcast_to(tv[j], (L,))

        @plsc.parallel_loop(0, HD)
        def _(d):
            t = texp[d, :]
            for k in range(KG):
                bufs[b, d, pl.ds(k * L, L)] += t

    start_in(0, 0)
    start_in(1, 1)

    def group(g, carry):
        for b in range(NBUF):
            i = g * NBUF + b
            bp = (b + 2) % NBUF

            @pl.when(i + 2 < NCHUNK)
            def _():
                @pl.when(i >= 2)
                def _():
                    wait_out(bp)
                start_in(i + 2, bp)

            wait_in(b)
            compute(i, b)
            start_out(i, b)
        return carry

    lax.fori_loop(0, NGRP, group, 0)
    for b in range(NBUF):
        wait_out(b)


def kernel(inputs, table):
    out_t = _col_add(jnp.transpose(inputs, (1, 2, 0)), table)
    return jnp.transpose(out_t, (2, 0, 1))
